# baseline (speedup 1.0000x reference)
import functools

import jax
import jax.numpy as jnp
from jax import lax
from jax.experimental import pallas as pl
from jax.experimental.pallas import tpu as pltpu
from jax.experimental.pallas import tpu_sc as plsc

NC = 2
NS = 16
NW = NC * NS
LANES = 128
CB = 64


def _build(A, V, D, B):
    assert B % NW == 0
    bpw = B // NW
    assert bpw % CB == 0
    nsub = bpw // CB
    ipw = bpw * A
    ipc = CB * A
    gpc = ipc // LANES
    assert ipc % LANES == 0
    apw = bpw // LANES
    assert bpw % LANES == 0

    mesh = plsc.VectorSubcoreMesh(
        core_axis_name="c", subcore_axis_name="s",
        num_cores=NC, num_subcores=NS)

    @functools.partial(
        pl.kernel,
        out_type=[
            jax.ShapeDtypeStruct((B * A, D), jnp.float32),
            jax.ShapeDtypeStruct((B, D), jnp.float32),
        ],
        mesh=mesh,
        compiler_params=pltpu.CompilerParams(use_tc_tiling_on_sc=False),
        scratch_types=[
            pltpu.VMEM((ipw,), jnp.int32),
            pltpu.VMEM((ipc,), jnp.int32),
            pltpu.VMEM((ipc, D), jnp.float32),
            pltpu.VMEM((bpw,), jnp.int32),
            pltpu.VMEM((bpw, D), jnp.float32),
            pltpu.SemaphoreType.DMA,
        ],
    )
    def lookup(mt_hbm, ma_hbm, off_hbm, tab_hbm, out_t_hbm, out_a_hbm,
               idx_v, off_v, rows_v, idx2_v, rows2_v, sem):
        wid = lax.axis_index("s") * NC + lax.axis_index("c")
        pltpu.sync_copy(off_hbm, off_v)
        pltpu.sync_copy(mt_hbm.at[pl.ds(wid * ipw, ipw)], idx_v)
        pltpu.sync_copy(ma_hbm.at[pl.ds(wid * bpw, bpw)], idx2_v)

        def fix(i, _):
            q = (i % (ipc // 16)) * 16
            sl = pl.ds(i * 16, 16)
            idx_v[sl] = idx_v[sl] + off_v[pl.ds(q, 16)]
            return 0
        lax.fori_loop(0, ipw // 16, fix, 0)

        def sub(k, _):
            for j in range(gpc):
                pltpu.make_async_copy(
                    tab_hbm.at[idx_v.at[pl.ds(k * ipc + j * LANES, LANES)]],
                    rows_v.at[pl.ds(j * LANES, LANES)], sem).start()
            for j in range(gpc):
                pltpu.make_async_copy(
                    tab_hbm.at[idx_v.at[pl.ds(k * ipc + j * LANES, LANES)]],
                    rows_v.at[pl.ds(j * LANES, LANES)], sem).wait()
            pltpu.sync_copy(rows_v, out_t_hbm.at[pl.ds(wid * ipw + k * ipc, ipc)])
            return 0
        lax.fori_loop(0, nsub, sub, 0)

        for j in range(apw):
            pltpu.make_async_copy(
                tab_hbm.at[idx2_v.at[pl.ds(j * LANES, LANES)]],
                rows2_v.at[pl.ds(j * LANES, LANES)], sem).start()
        for j in range(apw):
            pltpu.make_async_copy(
                tab_hbm.at[idx2_v.at[pl.ds(j * LANES, LANES)]],
                rows2_v.at[pl.ds(j * LANES, LANES)], sem).wait()
        pltpu.sync_copy(rows2_v, out_a_hbm.at[pl.ds(wid * bpw, bpw)])

    return lookup, ipc


def kernel(mask_tuple, mask_idx, mask_attrs, tables):
    B, A = mask_tuple.shape
    _, V, D = tables.shape
    lookup, ipc = _build(A, V, D, B)

    tab = tables.reshape(A * V, D)
    mt = mask_tuple.reshape(B * A)
    ma = (mask_attrs + mask_idx * V).astype(jnp.int32)
    off = (jnp.arange(ipc, dtype=jnp.int32) % A) * V

    out_t, out_a = lookup(mt, ma, off, tab)
    return out_t.reshape(B, A * D), out_a


# speedup vs baseline: 3.2920x; 3.2920x over previous
import functools

import jax
import jax.numpy as jnp
from jax import lax
from jax.experimental import pallas as pl
from jax.experimental.pallas import tpu as pltpu
from jax.experimental.pallas import tpu_sc as plsc

NC = 2
NS = 16
NW = NC * NS
HB = 8192


def _build(A, V, D, B):
    R = A * D
    assert R % NW == 0
    rpw = R // NW
    assert D == NW
    assert B % (2 * HB) == 0 or B == 2 * HB
    nh = B // HB

    mesh = plsc.VectorSubcoreMesh(
        core_axis_name="c", subcore_axis_name="s",
        num_cores=NC, num_subcores=NS)

    @functools.partial(
        pl.kernel,
        out_type=[
            jax.ShapeDtypeStruct((R, B), jnp.float32),
            jax.ShapeDtypeStruct((D, B), jnp.float32),
        ],
        mesh=mesh,
        compiler_params=pltpu.CompilerParams(
            use_tc_tiling_on_sc=True, needs_layout_passes=False),
        scratch_types=[
            pltpu.VMEM((V,), jnp.float32),
            pltpu.VMEM((HB,), jnp.int32),
            pltpu.VMEM((HB,), jnp.float32),
        ],
    )
    def lookup(mt_hbm, ma_hbm, tab_hbm, atab_hbm, out_t_hbm, out_a_hbm,
               row_v, idx_v, val_v):
        wid = lax.axis_index("s") * NC + lax.axis_index("c")

        def gather_half():
            def g(i, _):
                sl = pl.ds(i * 16, 16)
                val_v[sl] = plsc.load_gather(row_v, [idx_v[sl]])
                return 0
            lax.fori_loop(0, HB // 16, g, 0)

        def row_task(j, _):
            r = wid * rpw + j
            a = r // D
            pltpu.sync_copy(tab_hbm.at[a, r % D], row_v)
            for h in range(nh):
                pltpu.sync_copy(mt_hbm.at[a, pl.ds(h * HB, HB)], idx_v)
                gather_half()
                pltpu.sync_copy(val_v, out_t_hbm.at[r, pl.ds(h * HB, HB)])
            return 0
        lax.fori_loop(0, rpw, row_task, 0)

        pltpu.sync_copy(atab_hbm.at[wid], row_v)
        for h in range(nh):
            pltpu.sync_copy(ma_hbm.at[pl.ds(h * HB, HB)], idx_v)
            gather_half()
            pltpu.sync_copy(val_v, out_a_hbm.at[wid, pl.ds(h * HB, HB)])

    return lookup


def kernel(mask_tuple, mask_idx, mask_attrs, tables):
    B, A = mask_tuple.shape
    _, V, D = tables.shape
    lookup = _build(A, V, D, B)

    tab = tables.transpose(0, 2, 1)
    atab = lax.dynamic_index_in_dim(tab, mask_idx, 0, keepdims=False)
    mt = mask_tuple.T
    out_t, out_a = lookup(mt, mask_attrs, tab, atab)
    return out_t.T, out_a.T
